# baseline (speedup 1.0000x reference)
import functools

import jax
import jax.numpy as jnp
from jax import lax
from jax.experimental import pallas as pl
from jax.experimental.pallas import tpu as pltpu
from jax.experimental.pallas import tpu_sc as plsc

N = 10000
E = 320000
H = 128
R = 8

NC = 2
NS = 16
HC = H // NC

N_PAD = 10240
K = 96
E_PAD = 331776
GCHUNK = E_PAD // K
SCHUNK = GCHUNK // (NC * NS)
CW = GCHUNK // (NC * NS)
CT = R * N_PAD
CT_W = CT // NS
ROWS_W = N_PAD // NS
Y_ROWS = R * N_PAD
CRING = 4

_mesh = plsc.VectorSubcoreMesh(core_axis_name="c", subcore_axis_name="s")


def _mm_body(x_ref, w_ref, y_ref):
    y_ref[...] = jnp.dot(x_ref[...], w_ref[0],
                         preferred_element_type=jnp.float32).astype(jnp.bfloat16)


MMB = 2048


def _compute_y(x_pad, w8p):
    nbt = N_PAD // MMB
    return pl.pallas_call(
        _mm_body,
        grid=(nbt, R),
        in_specs=[
            pl.BlockSpec((MMB, H), lambda nb, r: (nb, 0)),
            pl.BlockSpec((1, H, H), lambda nb, r: (r, 0, 0)),
        ],
        out_specs=pl.BlockSpec((MMB, H), lambda nb, r: (r * nbt + nb, 0)),
        out_shape=jax.ShapeDtypeStruct((Y_ROWS, H), jnp.bfloat16),
    )(x_pad, w8p)


def _fill16(ref, nvec, val):
    def body(i, _):
        off = pl.multiple_of(i * 16, 16)
        ref[pl.ds(off, 16)] = jnp.full((16,), val, ref.dtype)
        return 0
    lax.fori_loop(0, nvec, body, 0)


@functools.partial(
    pl.kernel,
    out_type=jax.ShapeDtypeStruct((NC, CT), jnp.float32),
    mesh=_mesh,
    scratch_types=[
        pltpu.VMEM_SHARED((CT,), jnp.float32),
        pltpu.VMEM((CT_W,), jnp.float32),
        [pltpu.VMEM((3 * K,), jnp.int32)] * CRING,
        [pltpu.VMEM((K,), jnp.int32)] * CRING,
        pltpu.VMEM((K,), jnp.float32),
        [pltpu.SemaphoreType.DMA] * CRING,
    ],
    compiler_params=pltpu.CompilerParams(use_tc_tiling_on_sc=False),
)
def _count_kernel(ep_hbm, out_hbm, csh, zbuf, ebufs, skeys, onesb, sems):
    c = lax.axis_index("c")
    s = lax.axis_index("s")
    wid = s * NC + c

    _fill16(zbuf, CT_W // 16, 0.0)
    _fill16(onesb, K // 16, 1.0)
    zoff = pl.multiple_of(s * CT_W, 8)
    pltpu.sync_copy(zbuf, csh.at[pl.ds(zoff, CT_W)])
    plsc.subcore_barrier()

    def chunk(i, _):
        for b in range(CRING):
            idx = i * CRING + b

            @pl.when(idx >= CRING)
            def _():
                pltpu.make_async_copy(
                    onesb, csh.at[skeys[b]], sems[b]).wait()

            g = wid * CW + idx
            pltpu.sync_copy(ep_hbm.at[g], ebufs[b])
            for j in range(K // 16):
                sl = pl.ds(16 * j, 16)
                t16 = ebufs[b][pl.ds(2 * K + 16 * j, 16)]
                d16 = ebufs[b][pl.ds(K + 16 * j, 16)]
                skeys[b][sl] = t16 * N_PAD + d16
            pltpu.async_copy(onesb, csh.at[skeys[b]], sems[b], add=True)
        return 0

    lax.fori_loop(0, CW // CRING, chunk, 0)
    for b in range(CRING):
        pltpu.make_async_copy(onesb, csh.at[skeys[b]], sems[b]).wait()
    plsc.subcore_barrier()
    pltpu.sync_copy(csh.at[pl.ds(zoff, CT_W)], out_hbm.at[c, pl.ds(zoff, CT_W)])


@functools.partial(
    pl.kernel,
    out_type=jax.ShapeDtypeStruct((NC, N_PAD, H), jnp.float32),
    mesh=_mesh,
    scratch_types=[
        pltpu.VMEM_SHARED((N_PAD, H), jnp.float32),
        pltpu.VMEM_SHARED((CT,), jnp.float32),
        pltpu.VMEM((1280,), jnp.float32),
        pltpu.VMEM((1280,), jnp.float32),
        [pltpu.VMEM((3 * K,), jnp.int32)] * 2,
        [pltpu.VMEM((K,), jnp.int32)] * 2,
        [pltpu.VMEM((K,), jnp.int32)] * 2,
        [pltpu.VMEM((K,), jnp.int32)] * 2,
        [pltpu.VMEM((K,), jnp.float32)] * 2,
        [pltpu.VMEM((K, H), jnp.bfloat16)] * 2,
        [pltpu.VMEM((K, H), jnp.float32)] * 2,
        pltpu.VMEM((16, H), jnp.float32),
        [pltpu.SemaphoreType.DMA] * 2,
        [pltpu.SemaphoreType.DMA] * 2,
        [pltpu.SemaphoreType.DMA] * 2,
    ],
    compiler_params=pltpu.CompilerParams(needs_layout_passes=False,
                                         use_tc_tiling_on_sc=False),
)
def _scatter_kernel(ep_hbm, cnt_hbm, y_hbm, out_hbm,
                    acc, inv_sh, cb0, cb1, ebufs, gkeys, skeys, dsts, scales,
                    rowsb, rowf, zbuf, semG, semI, semS):
    c = lax.axis_index("c")
    s = lax.axis_index("s")

    for a in range(16):
        for j in range(H // 16):
            zbuf[a, pl.ds(16 * j, 16)] = jnp.zeros((16,), jnp.float32)

    def zrow(k, _):
        off = pl.multiple_of(s * ROWS_W + k * 16, 16)
        pltpu.sync_copy(zbuf, acc.at[pl.ds(off, 16)])
        return 0

    lax.fori_loop(0, ROWS_W // 16, zrow, 0)

    def invchunk(i, _):
        o = pl.multiple_of(s * CT_W + i * 1280, 8)
        pltpu.sync_copy(cnt_hbm.at[0, pl.ds(o, 1280)], cb0)
        pltpu.sync_copy(cnt_hbm.at[1, pl.ds(o, 1280)], cb1)

        def invvec(k, _):
            ko = pl.multiple_of(k * 16, 16)
            v = cb0[pl.ds(ko, 16)] + cb1[pl.ds(ko, 16)]
            cb0[pl.ds(ko, 16)] = 1.0 / jnp.maximum(v, 1.0)
            return 0

        lax.fori_loop(0, 80, invvec, 0)
        pltpu.sync_copy(cb0, inv_sh.at[pl.ds(o, 1280)])
        return 0

    lax.fori_loop(0, CT_W // 1280, invchunk, 0)
    plsc.subcore_barrier()

    def launch(idx, b):
        g = (c * NS + s) * SCHUNK + idx
        pltpu.sync_copy(ep_hbm.at[g], ebufs[b])
        for j in range(K // 16):
            sl = pl.ds(16 * j, 16)
            t16 = ebufs[b][pl.ds(2 * K + 16 * j, 16)]
            d16 = ebufs[b][pl.ds(K + 16 * j, 16)]
            gkeys[b][sl] = t16 * N_PAD + ebufs[b][sl]
            skeys[b][sl] = t16 * N_PAD + d16
            dsts[b][sl] = d16
        pltpu.async_copy(y_hbm.at[gkeys[b]], rowsb[b], semG[b])
        pltpu.async_copy(inv_sh.at[skeys[b]], scales[b], semI[b])

    launch(0, 0)

    def body(i, _):
        for b in range(2):
            idx = i * 2 + b
            nb = 1 - b

            @pl.when(idx + 1 < SCHUNK)
            def _():
                @pl.when(idx >= 1)
                def _():
                    pltpu.make_async_copy(
                        rowf[nb], acc.at[dsts[nb]], semS[nb]).wait()
                launch(idx + 1, nb)

            pltpu.make_async_copy(y_hbm.at[gkeys[b]], rowsb[b], semG[b]).wait()
            pltpu.make_async_copy(inv_sh.at[skeys[b]], scales[b],
                                  semI[b]).wait()
            for grp in range(K // 16):
                sv = scales[b][pl.ds(16 * grp, 16)]
                for e2 in range(16):
                    e = 16 * grp + e2
                    sc = sv[e2]
                    for j in range(H // 32):
                        v32 = rowsb[b][e, pl.ds(32 * j, 32)]
                        u = plsc.bitcast(v32, jnp.int32)
                        flo = plsc.bitcast(u << 16, jnp.float32)
                        fhi = plsc.bitcast(u & jnp.int32(-65536), jnp.float32)
                        rowf[b][e, pl.ds(16 * j, 16)] = flo * sc
                        rowf[b][e, pl.ds(HC + 16 * j, 16)] = fhi * sc
            pltpu.async_copy(rowf[b], acc.at[dsts[b]], semS[b],
                             add=True)
        return 0

    lax.fori_loop(0, SCHUNK // 2, body, 0)
    for b in range(2):
        pltpu.make_async_copy(rowf[b], acc.at[dsts[b]], semS[b]).wait()
    plsc.subcore_barrier()

    def frow(k, _):
        off = pl.multiple_of(s * ROWS_W + k * 16, 16)
        pltpu.sync_copy(acc.at[pl.ds(off, 16)], out_hbm.at[c, pl.ds(off, 16)])
        return 0

    lax.fori_loop(0, ROWS_W // 16, frow, 0)


def _fin_body(x_ref, wr_ref, acc_ref, bias_ref, g_ref, b_ref, o_ref):
    root = jnp.dot(x_ref[...], wr_ref[...], preferred_element_type=jnp.float32)
    v = root + acc_ref[0] + acc_ref[1] + bias_ref[0]
    mu = jnp.mean(v, axis=1, keepdims=True)
    d = v - mu
    var = jnp.mean(d * d, axis=1, keepdims=True)
    o_ref[...] = jnp.maximum(d * lax.rsqrt(var + 1e-5) * g_ref[0] + b_ref[0], 0.0)


def _finish(x_pad, w_root, acc, bias, ln_gamma, ln_beta):
    nbt = N_PAD // 512
    return pl.pallas_call(
        _fin_body,
        grid=(nbt,),
        in_specs=[
            pl.BlockSpec((512, H), lambda nb: (nb, 0)),
            pl.BlockSpec((H, H), lambda nb: (0, 0)),
            pl.BlockSpec((NC, 512, H), lambda nb: (0, nb, 0)),
            pl.BlockSpec((1, H), lambda nb: (0, 0)),
            pl.BlockSpec((1, H), lambda nb: (0, 0)),
            pl.BlockSpec((1, H), lambda nb: (0, 0)),
        ],
        out_specs=pl.BlockSpec((512, H), lambda nb: (nb, 0)),
        out_shape=jax.ShapeDtypeStruct((N_PAD, H), jnp.float32),
    )(x_pad, w_root, acc, bias, ln_gamma, ln_beta)


def kernel(x, edge_index, edge_type, W, W_root, bias, ln_gamma, ln_beta):
    src = edge_index[0].astype(jnp.int32)
    dst = edge_index[1].astype(jnp.int32)
    et = edge_type.astype(jnp.int32)
    pad = E_PAD - E
    src_p = jnp.concatenate([src, jnp.zeros((pad,), jnp.int32)])
    pad_dst = N + jnp.arange(pad, dtype=jnp.int32) % (N_PAD - N)
    dst_p = jnp.concatenate([dst, pad_dst])
    et_p = jnp.concatenate([et, jnp.zeros((pad,), jnp.int32)])
    epack = jnp.stack([src_p, dst_p, et_p]).reshape(3, GCHUNK, K)
    epack = epack.transpose(1, 0, 2).reshape(GCHUNK, 3 * K)
    x_pad = jnp.pad(x, ((0, N_PAD - N), (0, 0)))
    w8p = jnp.stack([W[:, :, :HC], W[:, :, HC:]], axis=-1).reshape(R, H, H)

    y = _compute_y(x_pad, w8p)
    counts = _count_kernel(epack)
    acc = _scatter_kernel(epack, counts, y)
    out = _finish(x_pad, W_root, acc, bias.reshape(1, H),
                  ln_gamma.reshape(1, H), ln_beta.reshape(1, H))
    return out[:N]


# speedup vs baseline: 1.0676x; 1.0676x over previous
import functools

import jax
import jax.numpy as jnp
from jax import lax
from jax.experimental import pallas as pl
from jax.experimental.pallas import tpu as pltpu
from jax.experimental.pallas import tpu_sc as plsc

N = 10000
E = 320000
H = 128
R = 8

NC = 2
NS = 16
HC = H // NC

N_PAD = 10240
K = 96
E_PAD = 331776
GCHUNK = E_PAD // K
SCHUNK = GCHUNK // (NC * NS)
CW = GCHUNK // (NC * NS)
CT = R * N_PAD
CT_W = CT // NS
ROWS_W = N_PAD // NS
Y_ROWS = R * N_PAD
CRING = 4

_mesh = plsc.VectorSubcoreMesh(core_axis_name="c", subcore_axis_name="s")


def _mm_body(x_ref, w_ref, y_ref):
    y_ref[...] = jnp.dot(x_ref[...], w_ref[0],
                         preferred_element_type=jnp.float32).astype(jnp.bfloat16)


MMB = 2048


def _compute_y(x_pad, w8p):
    nbt = N_PAD // MMB
    return pl.pallas_call(
        _mm_body,
        grid=(nbt, R),
        in_specs=[
            pl.BlockSpec((MMB, H), lambda nb, r: (nb, 0)),
            pl.BlockSpec((1, H, H), lambda nb, r: (r, 0, 0)),
        ],
        out_specs=pl.BlockSpec((MMB, H), lambda nb, r: (r * nbt + nb, 0)),
        out_shape=jax.ShapeDtypeStruct((Y_ROWS, H), jnp.bfloat16),
    )(x_pad, w8p)


def _fill16(ref, nvec, val):
    def body(i, _):
        off = pl.multiple_of(i * 16, 16)
        ref[pl.ds(off, 16)] = jnp.full((16,), val, ref.dtype)
        return 0
    lax.fori_loop(0, nvec, body, 0)


@functools.partial(
    pl.kernel,
    out_type=jax.ShapeDtypeStruct((NC, CT), jnp.float32),
    mesh=_mesh,
    scratch_types=[
        pltpu.VMEM_SHARED((CT,), jnp.float32),
        pltpu.VMEM((CT_W,), jnp.float32),
        [pltpu.VMEM((3 * K,), jnp.int32)] * CRING,
        [pltpu.VMEM((K,), jnp.int32)] * CRING,
        pltpu.VMEM((K,), jnp.float32),
        [pltpu.SemaphoreType.DMA] * CRING,
    ],
    compiler_params=pltpu.CompilerParams(use_tc_tiling_on_sc=False),
)
def _count_kernel(ep_hbm, out_hbm, csh, zbuf, ebufs, skeys, onesb, sems):
    c = lax.axis_index("c")
    s = lax.axis_index("s")
    wid = s * NC + c

    _fill16(zbuf, CT_W // 16, 0.0)
    _fill16(onesb, K // 16, 1.0)
    zoff = pl.multiple_of(s * CT_W, 8)
    pltpu.sync_copy(zbuf, csh.at[pl.ds(zoff, CT_W)])
    plsc.subcore_barrier()

    def chunk(i, _):
        for b in range(CRING):
            idx = i * CRING + b

            @pl.when(idx >= CRING)
            def _():
                pltpu.make_async_copy(
                    onesb, csh.at[skeys[b]], sems[b]).wait()

            g = wid * CW + idx
            pltpu.sync_copy(ep_hbm.at[g], ebufs[b])
            for j in range(K // 16):
                sl = pl.ds(16 * j, 16)
                t16 = ebufs[b][pl.ds(2 * K + 16 * j, 16)]
                d16 = ebufs[b][pl.ds(K + 16 * j, 16)]
                skeys[b][sl] = t16 * N_PAD + d16
            pltpu.async_copy(onesb, csh.at[skeys[b]], sems[b], add=True)
        return 0

    lax.fori_loop(0, CW // CRING, chunk, 0)
    for b in range(CRING):
        pltpu.make_async_copy(onesb, csh.at[skeys[b]], sems[b]).wait()
    plsc.subcore_barrier()
    pltpu.sync_copy(csh.at[pl.ds(zoff, CT_W)], out_hbm.at[c, pl.ds(zoff, CT_W)])


@functools.partial(
    pl.kernel,
    out_type=jax.ShapeDtypeStruct((NC, N_PAD, H), jnp.float32),
    mesh=_mesh,
    scratch_types=[
        pltpu.VMEM_SHARED((N_PAD, H), jnp.float32),
        pltpu.VMEM_SHARED((CT,), jnp.float32),
        pltpu.VMEM((1280,), jnp.float32),
        pltpu.VMEM((1280,), jnp.float32),
        [pltpu.VMEM((3 * K,), jnp.int32)] * 2,
        [pltpu.VMEM((K,), jnp.int32)] * 2,
        [pltpu.VMEM((K,), jnp.int32)] * 2,
        [pltpu.VMEM((K,), jnp.int32)] * 2,
        [pltpu.VMEM((K,), jnp.float32)] * 2,
        [pltpu.VMEM((K, H), jnp.bfloat16)] * 2,
        [pltpu.VMEM((K, H), jnp.float32)] * 2,
        pltpu.VMEM((16, H), jnp.float32),
        [pltpu.SemaphoreType.DMA] * 2,
        [pltpu.SemaphoreType.DMA] * 2,
        [pltpu.SemaphoreType.DMA] * 2,
    ],
    compiler_params=pltpu.CompilerParams(needs_layout_passes=False,
                                         use_tc_tiling_on_sc=False),
)
def _scatter_kernel(ep_hbm, cnt_hbm, y_hbm, out_hbm,
                    acc, inv_sh, cb0, cb1, ebufs, gkeys, skeys, dsts, scales,
                    rowsb, rowf, zbuf, semG, semI, semS):
    c = lax.axis_index("c")
    s = lax.axis_index("s")

    for a in range(16):
        for j in range(H // 16):
            zbuf[a, pl.ds(16 * j, 16)] = jnp.zeros((16,), jnp.float32)

    def zrow(k, _):
        off = pl.multiple_of(s * ROWS_W + k * 16, 16)
        pltpu.sync_copy(zbuf, acc.at[pl.ds(off, 16)])
        return 0

    lax.fori_loop(0, ROWS_W // 16, zrow, 0)

    def invchunk(i, _):
        o = pl.multiple_of(s * CT_W + i * 1280, 8)
        pltpu.sync_copy(cnt_hbm.at[0, pl.ds(o, 1280)], cb0)
        pltpu.sync_copy(cnt_hbm.at[1, pl.ds(o, 1280)], cb1)

        def invvec(k, _):
            ko = pl.multiple_of(k * 16, 16)
            v = cb0[pl.ds(ko, 16)] + cb1[pl.ds(ko, 16)]
            cb0[pl.ds(ko, 16)] = 1.0 / jnp.maximum(v, 1.0)
            return 0

        lax.fori_loop(0, 80, invvec, 0)
        pltpu.sync_copy(cb0, inv_sh.at[pl.ds(o, 1280)])
        return 0

    lax.fori_loop(0, CT_W // 1280, invchunk, 0)
    plsc.subcore_barrier()

    def launch(idx, b):
        g = (c * NS + s) * SCHUNK + idx
        pltpu.sync_copy(ep_hbm.at[g], ebufs[b])
        for j in range(K // 16):
            sl = pl.ds(16 * j, 16)
            t16 = ebufs[b][pl.ds(2 * K + 16 * j, 16)]
            d16 = ebufs[b][pl.ds(K + 16 * j, 16)]
            gkeys[b][sl] = t16 * N_PAD + ebufs[b][sl]
            skeys[b][sl] = t16 * N_PAD + d16
            dsts[b][sl] = d16
        pltpu.async_copy(y_hbm.at[gkeys[b]], rowsb[b], semG[b])
        pltpu.async_copy(inv_sh.at[skeys[b]], scales[b], semI[b])

    launch(0, 0)

    def body(i, _):
        for b in range(2):
            idx = i * 2 + b
            nb = 1 - b

            @pl.when(idx + 1 < SCHUNK)
            def _():
                @pl.when(idx >= 1)
                def _():
                    pltpu.make_async_copy(
                        rowf[nb], acc.at[dsts[nb]], semS[nb]).wait()
                launch(idx + 1, nb)

            pltpu.make_async_copy(y_hbm.at[gkeys[b]], rowsb[b], semG[b]).wait()
            pltpu.make_async_copy(inv_sh.at[skeys[b]], scales[b],
                                  semI[b]).wait()
            for grp in range(K // 16):
                sv = scales[b][pl.ds(16 * grp, 16)]
                for e2 in range(16):
                    e = 16 * grp + e2
                    sc = sv[e2]
                    for j in range(H // 32):
                        v32 = rowsb[b][e, pl.ds(32 * j, 32)]
                        u = plsc.bitcast(v32, jnp.int32)
                        flo = plsc.bitcast(u << 16, jnp.float32)
                        fhi = plsc.bitcast(u & jnp.int32(-65536), jnp.float32)
                        rowf[b][e, pl.ds(16 * j, 16)] = flo * sc
                        rowf[b][e, pl.ds(HC + 16 * j, 16)] = fhi * sc
            pltpu.async_copy(rowf[b], acc.at[dsts[b]], semS[b],
                             add=True)
        return 0

    lax.fori_loop(0, SCHUNK // 2, body, 0)
    for b in range(2):
        pltpu.make_async_copy(rowf[b], acc.at[dsts[b]], semS[b]).wait()
    plsc.subcore_barrier()

    def frow(k, _):
        off = pl.multiple_of(s * ROWS_W + k * 16, 16)
        pltpu.sync_copy(acc.at[pl.ds(off, 16)], out_hbm.at[c, pl.ds(off, 16)])
        return 0

    lax.fori_loop(0, ROWS_W // 16, frow, 0)


def _fin_body(x_ref, wr_ref, acc_ref, bias_ref, g_ref, b_ref, o_ref):
    root = jnp.dot(x_ref[...], wr_ref[...], preferred_element_type=jnp.float32)
    v = root + acc_ref[0] + acc_ref[1] + bias_ref[0]
    mu = jnp.mean(v, axis=1, keepdims=True)
    d = v - mu
    var = jnp.mean(d * d, axis=1, keepdims=True)
    o_ref[...] = jnp.maximum(d * lax.rsqrt(var + 1e-5) * g_ref[0] + b_ref[0], 0.0)


def _finish(x_pad, w_root, acc, bias, ln_gamma, ln_beta):
    nbt = N_PAD // 512
    return pl.pallas_call(
        _fin_body,
        grid=(nbt,),
        in_specs=[
            pl.BlockSpec((512, H), lambda nb: (nb, 0)),
            pl.BlockSpec((H, H), lambda nb: (0, 0)),
            pl.BlockSpec((NC, 512, H), lambda nb: (0, nb, 0)),
            pl.BlockSpec((1, H), lambda nb: (0, 0)),
            pl.BlockSpec((1, H), lambda nb: (0, 0)),
            pl.BlockSpec((1, H), lambda nb: (0, 0)),
        ],
        out_specs=pl.BlockSpec((512, H), lambda nb: (nb, 0)),
        out_shape=jax.ShapeDtypeStruct((N_PAD, H), jnp.float32),
    )(x_pad, w_root, acc, bias, ln_gamma, ln_beta)


def kernel(x, edge_index, edge_type, W, W_root, bias, ln_gamma, ln_beta):
    src = edge_index[0].astype(jnp.int32)
    dst = edge_index[1].astype(jnp.int32)
    et = edge_type.astype(jnp.int32)
    pad = E_PAD - E
    src_p = jnp.concatenate([src, jnp.zeros((pad,), jnp.int32)])
    pad_dst = N + jnp.arange(pad, dtype=jnp.int32) % (N_PAD - N)
    dst_p = jnp.concatenate([dst, pad_dst])
    et_p = jnp.concatenate([et, jnp.zeros((pad,), jnp.int32)])
    epack = jnp.stack([src_p, dst_p, et_p]).reshape(3, GCHUNK, K)
    epack = epack.transpose(1, 0, 2).reshape(GCHUNK, 3 * K)
    x_pad = jnp.pad(x, ((0, N_PAD - N), (0, 0)))
    col = jnp.arange(H, dtype=jnp.int32)
    perm = jnp.where(col % 2 == 0, col // 2, HC + col // 2)
    pmat = (perm[None, :] == col[:, None]).astype(jnp.float32)
    w8p = jnp.einsum('rik,km->rim', W, pmat)

    y = _compute_y(x_pad, w8p)
    counts = _count_kernel(epack)
    acc = _scatter_kernel(epack, counts, y)
    out = _finish(x_pad, W_root, acc, bias.reshape(1, H),
                  ln_gamma.reshape(1, H), ln_beta.reshape(1, H))
    return out[:N]


# speedup vs baseline: 1.2168x; 1.1398x over previous
import functools

import jax
import jax.numpy as jnp
from jax import lax
from jax.experimental import pallas as pl
from jax.experimental.pallas import tpu as pltpu
from jax.experimental.pallas import tpu_sc as plsc

N = 10000
E = 320000
H = 128
R = 8

NC = 2
NS = 16
HC = H // NC

N_PAD = 10240
K = 96
E_PAD = 331776
GCHUNK = E_PAD // K
SCHUNK = GCHUNK // (NC * NS)
CW = GCHUNK // (NC * NS)
CT = R * N_PAD
CT_W = CT // NS
ROWS_W = N_PAD // NS
Y_ROWS = R * N_PAD
CRING = 4

_mesh = plsc.VectorSubcoreMesh(core_axis_name="c", subcore_axis_name="s")


def _mm_body(x_ref, w_ref, y_ref):
    y_ref[...] = jnp.dot(x_ref[...], w_ref[0],
                         preferred_element_type=jnp.float32).astype(jnp.bfloat16)


MMB = 2048


def _compute_y(x_pad, w8p):
    nbt = N_PAD // MMB
    return pl.pallas_call(
        _mm_body,
        grid=(nbt, R),
        in_specs=[
            pl.BlockSpec((MMB, H), lambda nb, r: (nb, 0)),
            pl.BlockSpec((1, H, H), lambda nb, r: (r, 0, 0)),
        ],
        out_specs=pl.BlockSpec((MMB, H), lambda nb, r: (r * nbt + nb, 0)),
        out_shape=jax.ShapeDtypeStruct((Y_ROWS, H), jnp.bfloat16),
    )(x_pad, w8p)


def _fill16(ref, nvec, val):
    def body(i, _):
        off = pl.multiple_of(i * 16, 16)
        ref[pl.ds(off, 16)] = jnp.full((16,), val, ref.dtype)
        return 0
    lax.fori_loop(0, nvec, body, 0)


@functools.partial(
    pl.kernel,
    out_type=jax.ShapeDtypeStruct((NC, CT), jnp.float32),
    mesh=_mesh,
    scratch_types=[
        pltpu.VMEM_SHARED((CT,), jnp.float32),
        pltpu.VMEM((CT_W,), jnp.float32),
        [pltpu.VMEM((3 * K,), jnp.int32)] * CRING,
        [pltpu.VMEM((K,), jnp.int32)] * CRING,
        pltpu.VMEM((K,), jnp.float32),
        [pltpu.SemaphoreType.DMA] * CRING,
    ],
    compiler_params=pltpu.CompilerParams(use_tc_tiling_on_sc=False),
)
def _count_kernel(ep_hbm, out_hbm, csh, zbuf, ebufs, skeys, onesb, sems):
    c = lax.axis_index("c")
    s = lax.axis_index("s")
    wid = s * NC + c

    _fill16(zbuf, CT_W // 16, 0.0)
    _fill16(onesb, K // 16, 1.0)
    zoff = pl.multiple_of(s * CT_W, 8)
    pltpu.sync_copy(zbuf, csh.at[pl.ds(zoff, CT_W)])
    plsc.subcore_barrier()

    def chunk(i, _):
        for b in range(CRING):
            idx = i * CRING + b

            @pl.when(idx >= CRING)
            def _():
                pltpu.make_async_copy(
                    onesb, csh.at[skeys[b]], sems[b]).wait()

            g = wid * CW + idx
            pltpu.sync_copy(ep_hbm.at[g], ebufs[b])
            for j in range(K // 16):
                sl = pl.ds(16 * j, 16)
                t16 = ebufs[b][pl.ds(2 * K + 16 * j, 16)]
                d16 = ebufs[b][pl.ds(K + 16 * j, 16)]
                skeys[b][sl] = t16 * N_PAD + d16
            pltpu.async_copy(onesb, csh.at[skeys[b]], sems[b], add=True)
        return 0

    lax.fori_loop(0, CW // CRING, chunk, 0)
    for b in range(CRING):
        pltpu.make_async_copy(onesb, csh.at[skeys[b]], sems[b]).wait()
    plsc.subcore_barrier()
    pltpu.sync_copy(csh.at[pl.ds(zoff, CT_W)], out_hbm.at[c, pl.ds(zoff, CT_W)])


@functools.partial(
    pl.kernel,
    out_type=jax.ShapeDtypeStruct((NC, N_PAD, H), jnp.float32),
    mesh=_mesh,
    scratch_types=[
        pltpu.VMEM_SHARED((N_PAD, H), jnp.float32),
        pltpu.VMEM_SHARED((CT,), jnp.float32),
        pltpu.VMEM((1280,), jnp.float32),
        pltpu.VMEM((1280,), jnp.float32),
        [pltpu.VMEM((3 * K,), jnp.int32)] * 2,
        [pltpu.VMEM((K,), jnp.int32)] * 2,
        [pltpu.VMEM((K,), jnp.int32)] * 2,
        [pltpu.VMEM((K,), jnp.int32)] * 2,
        [pltpu.VMEM((K,), jnp.float32)] * 2,
        [pltpu.VMEM((K, H), jnp.bfloat16)] * 2,
        [pltpu.VMEM((K, H), jnp.float32)] * 2,
        pltpu.VMEM((16, H), jnp.float32),
        [pltpu.SemaphoreType.DMA] * 2,
        [pltpu.SemaphoreType.DMA] * 2,
        [pltpu.SemaphoreType.DMA] * 2,
    ],
    compiler_params=pltpu.CompilerParams(needs_layout_passes=False,
                                         use_tc_tiling_on_sc=False),
)
def _scatter_kernel(ep_hbm, cnt_hbm, y_hbm, out_hbm,
                    acc, inv_sh, cb0, cb1, ebufs, gkeys, skeys, dsts, scales,
                    rowsb, rowf, zbuf, semG, semI, semS):
    c = lax.axis_index("c")
    s = lax.axis_index("s")

    for a in range(16):
        for j in range(H // 16):
            zbuf[a, pl.ds(16 * j, 16)] = jnp.zeros((16,), jnp.float32)

    def zrow(k, _):
        off = pl.multiple_of(s * ROWS_W + k * 16, 16)
        pltpu.sync_copy(zbuf, acc.at[pl.ds(off, 16)])
        return 0

    lax.fori_loop(0, ROWS_W // 16, zrow, 0)

    def invchunk(i, _):
        o = pl.multiple_of(s * CT_W + i * 1280, 8)
        pltpu.sync_copy(cnt_hbm.at[0, pl.ds(o, 1280)], cb0)
        pltpu.sync_copy(cnt_hbm.at[1, pl.ds(o, 1280)], cb1)

        def invvec(k, _):
            ko = pl.multiple_of(k * 16, 16)
            v = cb0[pl.ds(ko, 16)] + cb1[pl.ds(ko, 16)]
            cb0[pl.ds(ko, 16)] = 1.0 / jnp.maximum(v, 1.0)
            return 0

        lax.fori_loop(0, 80, invvec, 0)
        pltpu.sync_copy(cb0, inv_sh.at[pl.ds(o, 1280)])
        return 0

    lax.fori_loop(0, CT_W // 1280, invchunk, 0)
    plsc.subcore_barrier()

    def launch(idx, b):
        g = (c * NS + s) * SCHUNK + idx
        pltpu.sync_copy(ep_hbm.at[g], ebufs[b])
        for j in range(K // 16):
            sl = pl.ds(16 * j, 16)
            t16 = ebufs[b][pl.ds(2 * K + 16 * j, 16)]
            d16 = ebufs[b][pl.ds(K + 16 * j, 16)]
            gkeys[b][sl] = t16 * N_PAD + ebufs[b][sl]
            skeys[b][sl] = t16 * N_PAD + d16
            dsts[b][sl] = d16
        pltpu.async_copy(y_hbm.at[gkeys[b]], rowsb[b], semG[b])
        pltpu.async_copy(inv_sh.at[skeys[b]], scales[b], semI[b])

    launch(0, 0)

    def body(i, _):
        for b in range(2):
            idx = i * 2 + b
            nb = 1 - b

            @pl.when(idx + 1 < SCHUNK)
            def _():
                @pl.when(idx >= 1)
                def _():
                    pltpu.make_async_copy(
                        rowf[nb], acc.at[dsts[nb]], semS[nb]).wait()
                launch(idx + 1, nb)

            pltpu.make_async_copy(y_hbm.at[gkeys[b]], rowsb[b], semG[b]).wait()
            pltpu.make_async_copy(inv_sh.at[skeys[b]], scales[b],
                                  semI[b]).wait()
            for grp in range(K // 16):
                sv = scales[b][pl.ds(16 * grp, 16)]
                for e2 in range(16):
                    e = 16 * grp + e2
                    sc = sv[e2]
                    for j in range(H // 32):
                        v32 = rowsb[b][e, pl.ds(32 * j, 32)]
                        u = plsc.bitcast(v32, jnp.int32)
                        flo = plsc.bitcast(u << 16, jnp.float32)
                        fhi = plsc.bitcast(u & jnp.int32(-65536), jnp.float32)
                        rowf[b][e, pl.ds(16 * j, 16)] = flo * sc
                        rowf[b][e, pl.ds(HC + 16 * j, 16)] = fhi * sc
            pltpu.async_copy(rowf[b], acc.at[dsts[b]], semS[b],
                             add=True)
        return 0

    lax.fori_loop(0, SCHUNK // 2, body, 0)
    for b in range(2):
        pltpu.make_async_copy(rowf[b], acc.at[dsts[b]], semS[b]).wait()
    plsc.subcore_barrier()

    def frow(k, _):
        off = pl.multiple_of(s * ROWS_W + k * 16, 16)
        pltpu.sync_copy(acc.at[pl.ds(off, 16)], out_hbm.at[c, pl.ds(off, 16)])
        return 0

    lax.fori_loop(0, ROWS_W // 16, frow, 0)


def _fin_body(x_ref, wr_ref, acc_ref, bias_ref, g_ref, b_ref, o_ref):
    root = jnp.dot(x_ref[...], wr_ref[...], preferred_element_type=jnp.float32)
    v = root + acc_ref[0] + acc_ref[1] + bias_ref[0]
    mu = jnp.mean(v, axis=1, keepdims=True)
    d = v - mu
    var = jnp.mean(d * d, axis=1, keepdims=True)
    o_ref[...] = jnp.maximum(d * lax.rsqrt(var + 1e-5) * g_ref[0] + b_ref[0], 0.0)


def _finish(x_pad, w_root, acc, bias, ln_gamma, ln_beta):
    nbt = N_PAD // 512
    return pl.pallas_call(
        _fin_body,
        grid=(nbt,),
        in_specs=[
            pl.BlockSpec((512, H), lambda nb: (nb, 0)),
            pl.BlockSpec((H, H), lambda nb: (0, 0)),
            pl.BlockSpec((NC, 512, H), lambda nb: (0, nb, 0)),
            pl.BlockSpec((1, H), lambda nb: (0, 0)),
            pl.BlockSpec((1, H), lambda nb: (0, 0)),
            pl.BlockSpec((1, H), lambda nb: (0, 0)),
        ],
        out_specs=pl.BlockSpec((512, H), lambda nb: (nb, 0)),
        out_shape=jax.ShapeDtypeStruct((N_PAD, H), jnp.float32),
    )(x_pad, w_root, acc, bias, ln_gamma, ln_beta)


def kernel(x, edge_index, edge_type, W, W_root, bias, ln_gamma, ln_beta):
    src = edge_index[0].astype(jnp.int32)
    dst = edge_index[1].astype(jnp.int32)
    et = edge_type.astype(jnp.int32)
    pad = E_PAD - E
    src_p = jnp.concatenate([src, jnp.zeros((pad,), jnp.int32)])
    pad_dst = N + jnp.arange(pad, dtype=jnp.int32) % (N_PAD - N)
    dst_p = jnp.concatenate([dst, pad_dst])
    et_p = jnp.concatenate([et, jnp.zeros((pad,), jnp.int32)])
    epack = jnp.stack([src_p, dst_p, et_p]).reshape(3, GCHUNK, K)
    epack = epack.transpose(1, 0, 2).reshape(GCHUNK, 3 * K)
    x_pad = jnp.pad(x, ((0, N_PAD - N), (0, 0)))
    col = jnp.arange(H, dtype=jnp.int32)
    perm = jnp.where(col % 2 == 0, col // 2, HC + col // 2)
    w8p = W[:, :, perm]

    y = _compute_y(x_pad, w8p)
    counts = _count_kernel(epack)
    acc = _scatter_kernel(epack, counts, y)
    out = _finish(x_pad, W_root, acc, bias.reshape(1, H),
                  ln_gamma.reshape(1, H), ln_beta.reshape(1, H))
    return out[:N]
